# Initial kernel scaffold; baseline (speedup 1.0000x reference)
import jax
import jax.numpy as jnp
from jax.experimental import pallas as pl


def kernel(x, edge_index, edge_attr, weight, bias):
    raise NotImplementedError("write your pallas kernel here")


# speedup vs baseline: 3.7366x; 3.7366x over previous
import functools

import jax
import jax.numpy as jnp
from jax import lax
from jax.experimental import pallas as pl
from jax.experimental.pallas import tpu as pltpu
from jax.experimental.pallas import tpu_sc as plsc

N_NODES = 10000
IN_CH = 128
OUT_CH = 128
N_EDGES = 320000

NC = 2
NS = 16
NW = NC * NS
EPW = N_EDGES // NW
CHUNK = 80
NCHUNK = EPW // CHUNK
ROWS_PER_TILE = N_NODES // NS
ZROWS = 125
NZC = ROWS_PER_TILE // ZROWS
ACC_W = IN_CH + 16
NV = IN_CH // 16


def _mm_body(x_ref, wt_ref, wb_ref, y1_ref, y2_ref):
    xb = x_ref[...]
    y1_ref[...] = jnp.dot(xb, wt_ref[...], preferred_element_type=jnp.float32)
    y2_ref[...] = jnp.dot(xb, wb_ref[...], preferred_element_type=jnp.float32)


def _combine_body(p_ref, y1_ref, b_ref, o_ref):
    p = p_ref[...]
    ps = p[0] + p[1]
    s1 = ps[:, IN_CH:IN_CH + 1]
    cnt = jnp.maximum(ps[:, IN_CH + 1:IN_CH + 2], 1.0)
    o_ref[...] = (ps[:, :IN_CH] + y1_ref[...] * s1) / cnt + b_ref[...]


def _edge_body(y2_hbm, idx_hbm, ea_hbm, part_hbm,
               idx_chunk, ea_chunk, rows_buf, sc_buf, acc):
    cid = lax.axis_index("c")
    sid = lax.axis_index("s")
    wid = cid * NS + sid

    def zrow(i, _):
        for v in range(ACC_W // 16):
            sc_buf[i, pl.ds(v * 16, 16)] = jnp.zeros((16,), jnp.float32)
        return 0
    lax.fori_loop(0, CHUNK, zrow, 0)
    NSTRIPE = N_NODES // CHUNK
    for j in range((NSTRIPE + NS - 1) // NS):
        st = sid + NS * j
        @pl.when(st < NSTRIPE)
        def _():
            pltpu.sync_copy(sc_buf, acc.at[pl.ds(st * CHUNK, CHUNK)])
    plsc.subcore_barrier()

    lanes = lax.iota(jnp.int32, 16)

    def chunk_body(c, _):
        pltpu.sync_copy(idx_hbm.at[wid, c], idx_chunk)
        pltpu.sync_copy(ea_hbm.at[wid, c], ea_chunk)
        pltpu.sync_copy(y2_hbm.at[idx_chunk.at[1]], rows_buf)

        def group_body(g, _):
            eav = ea_chunk[pl.ds(g * 16, 16)]
            base = g * 16
            for e16 in range(16):
                ea = eav[e16]
                e = base + e16
                for v in range(NV):
                    sc_buf[e, pl.ds(v * 16, 16)] = rows_buf[e, pl.ds(v * 16, 16)] * ea
                extra = jnp.where(lanes == 0, ea,
                                  jnp.where(lanes == 1, jnp.float32(1.0), jnp.float32(0.0)))
                sc_buf[e, pl.ds(IN_CH, 16)] = extra
            return 0
        lax.fori_loop(0, CHUNK // 16, group_body, 0)

        pltpu.sync_copy(sc_buf, acc.at[idx_chunk.at[0]], add=True)
        return 0
    lax.fori_loop(0, NCHUNK, chunk_body, 0)
    plsc.subcore_barrier()

    for j in range((NSTRIPE + NS - 1) // NS):
        st = sid + NS * j
        @pl.when(st < NSTRIPE)
        def _():
            pltpu.sync_copy(acc.at[pl.ds(st * CHUNK, CHUNK)], sc_buf)
            pltpu.sync_copy(sc_buf, part_hbm.at[cid, pl.ds(st * CHUNK, CHUNK)])


_edge_call = pl.kernel(
    _edge_body,
    out_type=jax.ShapeDtypeStruct((NC, N_NODES, ACC_W), jnp.float32),
    mesh=plsc.VectorSubcoreMesh(core_axis_name="c", subcore_axis_name="s",
                                num_cores=NC, num_subcores=NS),
    compiler_params=pltpu.CompilerParams(use_tc_tiling_on_sc=False),
    scratch_types=[
        pltpu.VMEM((2, CHUNK), jnp.int32),
        pltpu.VMEM((CHUNK,), jnp.float32),
        pltpu.VMEM((CHUNK, IN_CH), jnp.float32),
        pltpu.VMEM((CHUNK, ACC_W), jnp.float32),
        pltpu.VMEM_SHARED((N_NODES, ACC_W), jnp.float32),
    ],
)

MBLK = 2000

_mm_call = pl.pallas_call(
    _mm_body,
    grid=(N_NODES // MBLK,),
    in_specs=[
        pl.BlockSpec((MBLK, IN_CH), lambda i: (i, 0)),
        pl.BlockSpec((IN_CH, OUT_CH), lambda i: (0, 0)),
        pl.BlockSpec((IN_CH, OUT_CH), lambda i: (0, 0)),
    ],
    out_specs=[
        pl.BlockSpec((MBLK, OUT_CH), lambda i: (i, 0)),
        pl.BlockSpec((MBLK, OUT_CH), lambda i: (i, 0)),
    ],
    out_shape=[
        jax.ShapeDtypeStruct((N_NODES, OUT_CH), jnp.float32),
        jax.ShapeDtypeStruct((N_NODES, OUT_CH), jnp.float32),
    ],
)

CBLK = 2000

_combine_call = pl.pallas_call(
    _combine_body,
    grid=(N_NODES // CBLK,),
    in_specs=[
        pl.BlockSpec((NC, CBLK, ACC_W), lambda i: (0, i, 0)),
        pl.BlockSpec((CBLK, OUT_CH), lambda i: (i, 0)),
        pl.BlockSpec((1, OUT_CH), lambda i: (0, 0)),
    ],
    out_specs=pl.BlockSpec((CBLK, OUT_CH), lambda i: (i, 0)),
    out_shape=jax.ShapeDtypeStruct((N_NODES, OUT_CH), jnp.float32),
)


@jax.jit
def kernel(x, edge_index, edge_attr, weight, bias):
    ei = edge_index.astype(jnp.int32).reshape(2, NW, NCHUNK, CHUNK)
    idx2 = jnp.stack([ei[0], ei[1]], axis=2)
    ea = edge_attr.reshape(NW, NCHUNK, CHUNK)
    y1, y2 = _mm_call(x, weight[:IN_CH], weight[IN_CH:])
    part = _edge_call(y2, idx2, ea)
    return _combine_call(part, y1, bias.reshape(1, OUT_CH))


# speedup vs baseline: 10.3992x; 2.7831x over previous
import functools

import jax
import jax.numpy as jnp
from jax import lax
from jax.experimental import pallas as pl
from jax.experimental.pallas import tpu as pltpu
from jax.experimental.pallas import tpu_sc as plsc

N_NODES = 10000
IN_CH = 128
OUT_CH = 128
N_EDGES = 320000

NC = 2
NS = 16
NW = NC * NS
EPW = N_EDGES // NW
CHUNK = 80
NCHUNK = EPW // CHUNK
NV = IN_CH // 16
NSTRIPE = N_NODES // CHUNK
PERIOD = 6
NMAIN = (NCHUNK - PERIOD + 1) // PERIOD * PERIOD


def _mm_body(x_ref, wt_ref, wb_ref, y1_ref, y2_ref):
    xb = x_ref[...]
    y1_ref[...] = jnp.dot(xb, wt_ref[...], preferred_element_type=jnp.float32)
    y2_ref[...] = jnp.dot(xb, wb_ref[...], preferred_element_type=jnp.float32)


def _combine_body(pm_ref, pe_ref, y1_ref, b_ref, o_ref):
    pm = pm_ref[...]
    pe = pe_ref[...]
    pms = pm[0] + pm[1]
    pes = pe[0] + pe[1]
    s1 = pes[:, 0:1]
    cnt = jnp.maximum(pes[:, 1:2], 1.0)
    o_ref[...] = (pms + y1_ref[...] * s1) / cnt + b_ref[...]


def _edge_body(y2_hbm, idx_hbm, pm_hbm, pe_hbm,
               idx0, idx1, idx2, rows0, rows1, ex_buf,
               semi0, semi1, semi2, semg0, semg1, sems, semx,
               acc_m, acc_e):
    cid = lax.axis_index("c")
    sid = lax.axis_index("s")
    wid = cid * NS + sid
    idx = (idx0, idx1, idx2)
    semi = (semi0, semi1, semi2)
    rows = (rows0, rows1)
    semg = (semg0, semg1)

    def zrow(i, _):
        for v in range(NV):
            rows0[i, pl.ds(v * 16, 16)] = jnp.zeros((16,), jnp.float32)
        ex_buf[i, :] = jnp.zeros((16,), jnp.float32)
        return 0
    lax.fori_loop(0, CHUNK, zrow, 0)
    for j in range((NSTRIPE + NS - 1) // NS):
        st = sid + NS * j
        @pl.when(st < NSTRIPE)
        def _():
            pltpu.sync_copy(rows0, acc_m.at[pl.ds(st * CHUNK, CHUNK)])
            pltpu.sync_copy(ex_buf, acc_e.at[pl.ds(st * CHUNK, CHUNK)])
    plsc.subcore_barrier()

    lanes = lax.iota(jnp.int32, 16)

    def i_start(c, b3):
        pltpu.async_copy(idx_hbm.at[wid, c], idx[b3], semi[b3])

    def i_wait(c, b3):
        pltpu.make_async_copy(idx_hbm.at[wid, c], idx[b3], semi[b3]).wait()

    def g_start(b2, b3):
        pltpu.async_copy(y2_hbm.at[idx[b3].at[1]], rows[b2], semg[b2])

    def g_wait(b2, b3):
        pltpu.make_async_copy(y2_hbm.at[idx[b3].at[1]], rows[b2],
                              semg[b2]).wait()

    def compute(b2, b3):
        rbuf = idx[b3]
        rows_b = rows[b2]

        def group_body(g, _):
            eav = plsc.bitcast(rbuf[2, pl.ds(g * 16, 16)], jnp.float32)
            base = g * 16
            for e16 in range(16):
                ea = eav[e16]
                e = base + e16
                for v in range(NV):
                    rows_b[e, pl.ds(v * 16, 16)] = rows_b[e, pl.ds(v * 16, 16)] * ea
                ex_buf[e, :] = jnp.where(
                    lanes == 0, ea,
                    jnp.where(lanes == 1, jnp.float32(1.0), jnp.float32(0.0)))
            return 0
        lax.fori_loop(0, CHUNK // 16, group_body, 0)

    def s_start(b2, b3):
        dm = pltpu.async_copy(rows[b2], acc_m.at[idx[b3].at[0]], sems, add=True)
        de = pltpu.async_copy(ex_buf, acc_e.at[idx[b3].at[0]], semx, add=True)
        return dm, de

    def step(c, b2, b3, has_next, has_pf):
        g_wait(b2, b3)
        if has_next:
            nb2, nb3 = 1 - b2, (b3 + 1) % 3
            i_wait(c + 1, nb3)
            g_start(nb2, nb3)
        compute(b2, b3)
        dm, de = s_start(b2, b3)
        dm.wait()
        de.wait()
        if has_pf:
            i_start(c + 3, b3)

    for k in range(3):
        i_start(k, k)
    i_wait(0, 0)
    g_start(0, 0)

    def main_body(i, _):
        c0 = i * PERIOD
        for k in range(PERIOD):
            step(c0 + k, k % 2, k % 3, True, True)
        return 0
    lax.fori_loop(0, NMAIN // PERIOD, main_body, 0)

    for c in range(NMAIN, NCHUNK):
        step(c, c % 2, c % 3, c + 1 < NCHUNK, c + 3 < NCHUNK)

    plsc.subcore_barrier()

    for j in range((NSTRIPE + NS - 1) // NS):
        st = sid + NS * j
        @pl.when(st < NSTRIPE)
        def _():
            pltpu.sync_copy(acc_m.at[pl.ds(st * CHUNK, CHUNK)], rows0)
            pltpu.sync_copy(rows0, pm_hbm.at[cid, pl.ds(st * CHUNK, CHUNK)])
            pltpu.sync_copy(acc_e.at[pl.ds(st * CHUNK, CHUNK)], ex_buf)
            pltpu.sync_copy(ex_buf, pe_hbm.at[cid, pl.ds(st * CHUNK, CHUNK)])


_edge_call = pl.kernel(
    _edge_body,
    out_type=[
        jax.ShapeDtypeStruct((NC, N_NODES, IN_CH), jnp.float32),
        jax.ShapeDtypeStruct((NC, N_NODES, 16), jnp.float32),
    ],
    mesh=plsc.VectorSubcoreMesh(core_axis_name="c", subcore_axis_name="s",
                                num_cores=NC, num_subcores=NS),
    compiler_params=pltpu.CompilerParams(use_tc_tiling_on_sc=False,
                                         needs_layout_passes=False),
    scratch_types=[
        pltpu.VMEM((3, CHUNK), jnp.int32),
        pltpu.VMEM((3, CHUNK), jnp.int32),
        pltpu.VMEM((3, CHUNK), jnp.int32),
        pltpu.VMEM((CHUNK, IN_CH), jnp.float32),
        pltpu.VMEM((CHUNK, IN_CH), jnp.float32),
        pltpu.VMEM((CHUNK, 16), jnp.float32),
        pltpu.SemaphoreType.DMA,
        pltpu.SemaphoreType.DMA,
        pltpu.SemaphoreType.DMA,
        pltpu.SemaphoreType.DMA,
        pltpu.SemaphoreType.DMA,
        pltpu.SemaphoreType.DMA,
        pltpu.SemaphoreType.DMA,
        pltpu.VMEM_SHARED((N_NODES, IN_CH), jnp.float32),
        pltpu.VMEM_SHARED((N_NODES, 16), jnp.float32),
    ],
)

MBLK = 2000

_mm_call = pl.pallas_call(
    _mm_body,
    grid=(N_NODES // MBLK,),
    in_specs=[
        pl.BlockSpec((MBLK, IN_CH), lambda i: (i, 0)),
        pl.BlockSpec((IN_CH, OUT_CH), lambda i: (0, 0)),
        pl.BlockSpec((IN_CH, OUT_CH), lambda i: (0, 0)),
    ],
    out_specs=[
        pl.BlockSpec((MBLK, OUT_CH), lambda i: (i, 0)),
        pl.BlockSpec((MBLK, OUT_CH), lambda i: (i, 0)),
    ],
    out_shape=[
        jax.ShapeDtypeStruct((N_NODES, OUT_CH), jnp.float32),
        jax.ShapeDtypeStruct((N_NODES, OUT_CH), jnp.float32),
    ],
)

CBLK = 2000

_combine_call = pl.pallas_call(
    _combine_body,
    grid=(N_NODES // CBLK,),
    in_specs=[
        pl.BlockSpec((NC, CBLK, IN_CH), lambda i: (0, i, 0)),
        pl.BlockSpec((NC, CBLK, 16), lambda i: (0, i, 0)),
        pl.BlockSpec((CBLK, OUT_CH), lambda i: (i, 0)),
        pl.BlockSpec((1, OUT_CH), lambda i: (0, 0)),
    ],
    out_specs=pl.BlockSpec((CBLK, OUT_CH), lambda i: (i, 0)),
    out_shape=jax.ShapeDtypeStruct((N_NODES, OUT_CH), jnp.float32),
)


@jax.jit
def kernel(x, edge_index, edge_attr, weight, bias):
    ei = edge_index.astype(jnp.int32).reshape(2, NW, NCHUNK, CHUNK)
    eab = lax.bitcast_convert_type(edge_attr, jnp.int32).reshape(NW, NCHUNK, CHUNK)
    idx3 = jnp.stack([ei[0], ei[1], eab], axis=2)
    y1, y2 = _mm_call(x, weight[:IN_CH], weight[IN_CH:])
    pm, pe = _edge_call(y2, idx3)
    return _combine_call(pm, pe, y1, bias.reshape(1, OUT_CH))


# speedup vs baseline: 10.7602x; 1.0347x over previous
import functools

import jax
import jax.numpy as jnp
from jax import lax
from jax.experimental import pallas as pl
from jax.experimental.pallas import tpu as pltpu
from jax.experimental.pallas import tpu_sc as plsc

N_NODES = 10000
IN_CH = 128
OUT_CH = 128
N_EDGES = 320000

NC = 2
NS = 16
NW = NC * NS
EPW = N_EDGES // NW
CHUNK = 80
NCHUNK = EPW // CHUNK
NV = IN_CH // 16
NSTRIPE = N_NODES // CHUNK
PERIOD = 6


def _final_body(x_ref, z_ref, pe_ref, wt_ref, wb_ref, b_ref, o_ref):
    z = z_ref[...]
    zs = z[0] + z[1]
    pe = pe_ref[...]
    pes = pe[0] + pe[1]
    s1 = pes[:, 0:1]
    cnt = jnp.maximum(pes[:, 1:2], 1.0)
    y1 = jnp.dot(x_ref[...], wt_ref[...], preferred_element_type=jnp.float32)
    s2 = jnp.dot(zs, wb_ref[...], preferred_element_type=jnp.float32)
    o_ref[...] = (s2 + y1 * s1) / cnt + b_ref[...]


def _edge_body(x_hbm, idx_hbm, pm_hbm, pe_hbm,
               idx0, idx1, idx2, rows0, rows1, rows2, ex0, ex1,
               semi0, semi1, semi2, semg0, semg1, semg2,
               semm0, semm1, seme0, seme1, semz,
               acc_m, acc_e):
    cid = lax.axis_index("c")
    sid = lax.axis_index("s")
    wid = cid * NS + sid
    idx = (idx0, idx1, idx2)
    semi = (semi0, semi1, semi2)
    rows = (rows0, rows1, rows2)
    semg = (semg0, semg1, semg2)
    ex = (ex0, ex1)
    semm = (semm0, semm1)
    seme = (seme0, seme1)

    def i_start(c, s):
        pltpu.async_copy(idx_hbm.at[wid, c], idx[s], semi[s])

    def i_wait(c, s):
        pltpu.make_async_copy(idx_hbm.at[wid, c], idx[s], semi[s]).wait()

    def g_start(s):
        pltpu.async_copy(x_hbm.at[idx[s].at[1]], rows[s], semg[s])

    def g_wait(s):
        pltpu.make_async_copy(x_hbm.at[idx[s].at[1]], rows[s], semg[s]).wait()

    lanes = lax.iota(jnp.int32, 16)

    def compute(s, x2):
        rbuf = idx[s]
        rows_b = rows[s]
        ex_b = ex[x2]

        def group_body(g, _):
            eav = plsc.bitcast(rbuf[2, pl.ds(g * 16, 16)], jnp.float32)
            base = g * 16
            for e16 in range(16):
                ea = eav[e16]
                e = base + e16
                for v in range(NV):
                    rows_b[e, pl.ds(v * 16, 16)] = rows_b[e, pl.ds(v * 16, 16)] * ea
                ex_b[e, :] = jnp.where(
                    lanes == 0, ea,
                    jnp.where(lanes == 1, jnp.float32(1.0), jnp.float32(0.0)))
            return 0
        lax.fori_loop(0, CHUNK // 16, group_body, 0)

    def s_start(s, x2):
        pltpu.async_copy(rows[s], acc_m.at[idx[s].at[0]], semm[x2], add=True)
        pltpu.async_copy(ex[x2], acc_e.at[idx[s].at[0]], seme[x2], add=True)

    def s_wait(s, x2):
        pltpu.make_async_copy(rows[s], acc_m.at[idx[s].at[0]], semm[x2]).wait()
        pltpu.make_async_copy(ex[x2], acc_e.at[idx[s].at[0]], seme[x2]).wait()

    def step(c, s, x2, has_next, has_pf, has_prev):
        g_wait(s)
        if has_next:
            i_wait(c + 1, (s + 1) % 3)
            g_start((s + 1) % 3)
        compute(s, x2)
        s_start(s, x2)
        if has_prev:
            s_wait((s + 2) % 3, 1 - x2)
        if has_pf:
            i_start(c + 2, (s + 2) % 3)

    i_start(0, 0)
    i_start(1, 1)

    def zrow(i, _):
        for v in range(NV):
            rows0[i, pl.ds(v * 16, 16)] = jnp.zeros((16,), jnp.float32)
        ex0[i, :] = jnp.zeros((16,), jnp.float32)
        return 0
    lax.fori_loop(0, CHUNK, zrow, 0)
    NJ = (NSTRIPE + NS - 1) // NS
    for j in range(NJ):
        st = sid + NS * j
        @pl.when(st < NSTRIPE)
        def _():
            pltpu.async_copy(rows0, acc_m.at[pl.ds(st * CHUNK, CHUNK)], semz)
            pltpu.async_copy(ex0, acc_e.at[pl.ds(st * CHUNK, CHUNK)], semz)
    for j in range(NJ):
        st = sid + NS * j
        @pl.when(st < NSTRIPE)
        def _():
            pltpu.make_async_copy(rows0, acc_m.at[pl.ds(st * CHUNK, CHUNK)], semz).wait()
            pltpu.make_async_copy(ex0, acc_e.at[pl.ds(st * CHUNK, CHUNK)], semz).wait()
    plsc.subcore_barrier()

    i_wait(0, 0)
    g_start(0)
    step(0, 0, 0, True, True, False)

    def main_body(i, _):
        c0 = i * PERIOD + 1
        for k in range(PERIOD):
            ck = k + 1
            step(c0 + k, ck % 3, ck % 2, True, True, True)
        return 0
    NMAIN = (NCHUNK - 1 - 4) // PERIOD * PERIOD
    lax.fori_loop(0, NMAIN // PERIOD, main_body, 0)

    for c in range(NMAIN + 1, NCHUNK):
        step(c, c % 3, c % 2, c + 1 < NCHUNK, c + 2 < NCHUNK, True)
    s_wait((NCHUNK - 1) % 3, (NCHUNK - 1) % 2)
    plsc.subcore_barrier()

    for j in range(NJ):
        st = sid + NS * j
        @pl.when(st < NSTRIPE)
        def _():
            pltpu.async_copy(acc_m.at[pl.ds(st * CHUNK, CHUNK)],
                             pm_hbm.at[cid, pl.ds(st * CHUNK, CHUNK)], semz)
            pltpu.async_copy(acc_e.at[pl.ds(st * CHUNK, CHUNK)],
                             pe_hbm.at[cid, pl.ds(st * CHUNK, CHUNK)], semz)
    for j in range(NJ):
        st = sid + NS * j
        @pl.when(st < NSTRIPE)
        def _():
            pltpu.make_async_copy(acc_m.at[pl.ds(st * CHUNK, CHUNK)],
                                  pm_hbm.at[cid, pl.ds(st * CHUNK, CHUNK)], semz).wait()
            pltpu.make_async_copy(acc_e.at[pl.ds(st * CHUNK, CHUNK)],
                                  pe_hbm.at[cid, pl.ds(st * CHUNK, CHUNK)], semz).wait()


_edge_call = pl.kernel(
    _edge_body,
    out_type=[
        jax.ShapeDtypeStruct((NC, N_NODES, IN_CH), jnp.float32),
        jax.ShapeDtypeStruct((NC, N_NODES, 16), jnp.float32),
    ],
    mesh=plsc.VectorSubcoreMesh(core_axis_name="c", subcore_axis_name="s",
                                num_cores=NC, num_subcores=NS),
    compiler_params=pltpu.CompilerParams(use_tc_tiling_on_sc=False,
                                         needs_layout_passes=False),
    scratch_types=[
        pltpu.VMEM((3, CHUNK), jnp.int32),
        pltpu.VMEM((3, CHUNK), jnp.int32),
        pltpu.VMEM((3, CHUNK), jnp.int32),
        pltpu.VMEM((CHUNK, IN_CH), jnp.float32),
        pltpu.VMEM((CHUNK, IN_CH), jnp.float32),
        pltpu.VMEM((CHUNK, IN_CH), jnp.float32),
        pltpu.VMEM((CHUNK, 16), jnp.float32),
        pltpu.VMEM((CHUNK, 16), jnp.float32),
        pltpu.SemaphoreType.DMA,
        pltpu.SemaphoreType.DMA,
        pltpu.SemaphoreType.DMA,
        pltpu.SemaphoreType.DMA,
        pltpu.SemaphoreType.DMA,
        pltpu.SemaphoreType.DMA,
        pltpu.SemaphoreType.DMA,
        pltpu.SemaphoreType.DMA,
        pltpu.SemaphoreType.DMA,
        pltpu.SemaphoreType.DMA,
        pltpu.SemaphoreType.DMA,
        pltpu.VMEM_SHARED((N_NODES, IN_CH), jnp.float32),
        pltpu.VMEM_SHARED((N_NODES, 16), jnp.float32),
    ],
)

FBLK = 2000

_final_call = pl.pallas_call(
    _final_body,
    grid=(N_NODES // FBLK,),
    in_specs=[
        pl.BlockSpec((FBLK, IN_CH), lambda i: (i, 0)),
        pl.BlockSpec((NC, FBLK, IN_CH), lambda i: (0, i, 0)),
        pl.BlockSpec((NC, FBLK, 16), lambda i: (0, i, 0)),
        pl.BlockSpec((IN_CH, OUT_CH), lambda i: (0, 0)),
        pl.BlockSpec((IN_CH, OUT_CH), lambda i: (0, 0)),
        pl.BlockSpec((1, OUT_CH), lambda i: (0, 0)),
    ],
    out_specs=pl.BlockSpec((FBLK, OUT_CH), lambda i: (i, 0)),
    out_shape=jax.ShapeDtypeStruct((N_NODES, OUT_CH), jnp.float32),
)


@jax.jit
def kernel(x, edge_index, edge_attr, weight, bias):
    ei = edge_index.astype(jnp.int32).reshape(2, NW, NCHUNK, CHUNK)
    eab = lax.bitcast_convert_type(edge_attr, jnp.int32).reshape(NW, NCHUNK, CHUNK)
    idx3 = jnp.stack([ei[0], ei[1], eab], axis=2)
    zm, pe = _edge_call(x, idx3)
    return _final_call(x, zm, pe, weight[:IN_CH], weight[IN_CH:],
                       bias.reshape(1, OUT_CH))
